# baseline (speedup 1.0000x reference)
import jax
import jax.numpy as jnp
from jax.experimental import pallas as pl
from jax.experimental.pallas import tpu as pltpu

B, T, IN, H = 64, 128, 1024, 2048
FH = 4 * H
TB = T * B
U = 2
RT = 8


def _gemm_bias_kernel(x_ref, w_ref, b_ref, o_ref):
    o_ref[...] = (
        jnp.dot(x_ref[...], w_ref[...], preferred_element_type=jnp.float32)
        + b_ref[...]
    ).astype(jnp.bfloat16)


def _lstm_step_kernel(yx_ref, wat_ref, a0_ref, c0_ref,
                      out_ref, a_scr, c_scr):
    t = pl.program_id(0)

    @pl.when(t == 0)
    def _init():
        a_scr[...] = a0_ref[...]
        c_scr[...] = c0_ref[...]

    a_prev = a_scr[...]
    c_prev = c_scr[...]
    for s in range(U):
        y = yx_ref[s * B:(s + 1) * B, :] + jnp.dot(
            a_prev.astype(jnp.bfloat16), wat_ref[...],
            preferred_element_type=jnp.float32)
        yi = y[:, 0 * H:1 * H]
        yf = y[:, 1 * H:2 * H]
        yg = y[:, 2 * H:3 * H]
        yo = y[:, 3 * H:4 * H]
        c_t = jax.nn.sigmoid(yf) * c_prev + jax.nn.sigmoid(yi) * jnp.tanh(yg)
        a_t = jax.nn.sigmoid(yo) * jnp.tanh(c_t)
        hs = slice(s * H, (s + 1) * H)
        out_ref[0, :, hs] = a_t.astype(jnp.bfloat16)
        out_ref[1, :, hs] = c_t.astype(jnp.bfloat16)
        out_ref[2, :, hs] = yi.astype(jnp.bfloat16)
        out_ref[3, :, hs] = yf.astype(jnp.bfloat16)
        out_ref[4, :, hs] = yg.astype(jnp.bfloat16)
        out_ref[5, :, hs] = yo.astype(jnp.bfloat16)
        a_prev, c_prev = a_t, c_t
    a_scr[...] = a_prev
    c_scr[...] = c_prev


def _relayout_kernel(x0_ref, x1_ref, o0_ref, o1_ref):
    o0_ref[...] = x0_ref[0].reshape(B, RT, H).astype(jnp.float32)
    o1_ref[...] = x1_ref[0].reshape(B, RT, H).astype(jnp.float32)


def _to_bth_tc(packed):
    outs = []
    gate_sd = jax.ShapeDtypeStruct((B, T, H), jnp.float32)
    for g in range(0, 6, 2):
        res = pl.pallas_call(
            _relayout_kernel,
            grid=(T // RT,),
            in_specs=[
                pl.BlockSpec((1, B, RT * H), lambda i, g=g: (g, 0, i)),
                pl.BlockSpec((1, B, RT * H), lambda i, g=g: (g + 1, 0, i)),
            ],
            out_specs=[pl.BlockSpec((B, RT, H), lambda i: (0, i, 0))] * 2,
            out_shape=[gate_sd] * 2,
            compiler_params=pltpu.CompilerParams(
                dimension_semantics=("arbitrary",)),
        )(packed, packed)
        outs.extend(res)
    return outs


@jax.jit
def kernel(x, Wx, Wa, b, a0, c0):
    x_tm = jnp.swapaxes(x, 0, 1).reshape(TB, IN).astype(jnp.bfloat16)
    wxt = Wx.T.astype(jnp.bfloat16)
    b2 = b.reshape(1, FH)

    BM, BN = 2048, 2048
    yx = pl.pallas_call(
        _gemm_bias_kernel,
        grid=(TB // BM, FH // BN),
        in_specs=[
            pl.BlockSpec((BM, IN), lambda i, j: (i, 0)),
            pl.BlockSpec((IN, BN), lambda i, j: (0, j)),
            pl.BlockSpec((1, BN), lambda i, j: (0, j)),
        ],
        out_specs=pl.BlockSpec((BM, BN), lambda i, j: (i, j)),
        out_shape=jax.ShapeDtypeStruct((TB, FH), jnp.bfloat16),
        compiler_params=pltpu.CompilerParams(
            dimension_semantics=("parallel", "arbitrary")),
    )(x_tm, wxt, b2)

    wat = Wa.T.astype(jnp.bfloat16)

    packed = pl.pallas_call(
        _lstm_step_kernel,
        grid=(T // U,),
        in_specs=[
            pl.BlockSpec((U * B, FH), lambda t: (t, 0)),
            pl.BlockSpec((H, FH), lambda t: (0, 0)),
            pl.BlockSpec((B, H), lambda t: (0, 0)),
            pl.BlockSpec((B, H), lambda t: (0, 0)),
        ],
        out_specs=pl.BlockSpec((6, B, U * H), lambda t: (0, 0, t)),
        out_shape=jax.ShapeDtypeStruct((6, B, T * H), jnp.bfloat16),
        scratch_shapes=[
            pltpu.VMEM((B, H), jnp.float32),
            pltpu.VMEM((B, H), jnp.float32),
        ],
        compiler_params=pltpu.CompilerParams(
            dimension_semantics=("arbitrary",)),
    )(yx, wat, a0, c0)

    a, c, yi, yf, yg, yo = _to_bth_tc(packed)
    return (a, c, yi, yf, yg, yo)


# speedup vs baseline: 1.0024x; 1.0024x over previous
import jax
import jax.numpy as jnp
from jax.experimental import pallas as pl
from jax.experimental.pallas import tpu as pltpu

B, T, IN, H = 64, 128, 1024, 2048
FH = 4 * H
TB = T * B
U = 2
RT = 8


def _gemm_bias_kernel(x_ref, w_ref, b_ref, o_ref):
    o_ref[...] = (
        jnp.dot(x_ref[...], w_ref[...], preferred_element_type=jnp.float32)
        + b_ref[...]
    ).astype(jnp.bfloat16)


def _lstm_step_kernel(yx_ref, wat_ref, a0_ref, c0_ref,
                      a_out, c_out, yi_out, yf_out, yg_out, yo_out,
                      a_scr, c_scr):
    t = pl.program_id(0)

    @pl.when(t == 0)
    def _init():
        a_scr[...] = a0_ref[...]
        c_scr[...] = c0_ref[...]

    a_prev = a_scr[...]
    c_prev = c_scr[...]
    for s in range(U):
        y = yx_ref[s * B:(s + 1) * B, :] + jnp.dot(
            a_prev.astype(jnp.bfloat16), wat_ref[...],
            preferred_element_type=jnp.float32)
        yi = y[:, 0 * H:1 * H]
        yf = y[:, 1 * H:2 * H]
        yg = y[:, 2 * H:3 * H]
        yo = y[:, 3 * H:4 * H]
        c_t = jax.nn.sigmoid(yf) * c_prev + jax.nn.sigmoid(yi) * jnp.tanh(yg)
        a_t = jax.nn.sigmoid(yo) * jnp.tanh(c_t)
        hs = slice(s * H, (s + 1) * H)
        yi_out[:, hs] = yi.astype(jnp.bfloat16)
        yf_out[:, hs] = yf.astype(jnp.bfloat16)
        yg_out[:, hs] = yg.astype(jnp.bfloat16)
        yo_out[:, hs] = yo.astype(jnp.bfloat16)
        c_out[:, hs] = c_t.astype(jnp.bfloat16)
        a_out[:, hs] = a_t.astype(jnp.bfloat16)
        a_prev, c_prev = a_t, c_t
    a_scr[...] = a_prev
    c_scr[...] = c_prev


def _relayout_kernel(x0_ref, x1_ref, o0_ref, o1_ref):
    o0_ref[...] = x0_ref[...].reshape(B, RT, H).astype(jnp.float32)
    o1_ref[...] = x1_ref[...].reshape(B, RT, H).astype(jnp.float32)


def _to_bth_tc(flats):
    outs = []
    gate_sd = jax.ShapeDtypeStruct((B, T, H), jnp.float32)
    for k in range(0, 6, 2):
        res = pl.pallas_call(
            _relayout_kernel,
            grid=(T // RT,),
            in_specs=[pl.BlockSpec((B, RT * H), lambda i: (0, i))] * 2,
            out_specs=[pl.BlockSpec((B, RT, H), lambda i: (0, i, 0))] * 2,
            out_shape=[gate_sd] * 2,
            compiler_params=pltpu.CompilerParams(
                dimension_semantics=("arbitrary",)),
        )(*flats[k:k + 2])
        outs.extend(res)
    return outs


@jax.jit
def kernel(x, Wx, Wa, b, a0, c0):
    x_tm = jnp.swapaxes(x, 0, 1).reshape(TB, IN).astype(jnp.bfloat16)
    wxt = Wx.T.astype(jnp.bfloat16)
    b2 = b.reshape(1, FH)

    BM, BN = 2048, 2048
    yx = pl.pallas_call(
        _gemm_bias_kernel,
        grid=(TB // BM, FH // BN),
        in_specs=[
            pl.BlockSpec((BM, IN), lambda i, j: (i, 0)),
            pl.BlockSpec((IN, BN), lambda i, j: (0, j)),
            pl.BlockSpec((1, BN), lambda i, j: (0, j)),
        ],
        out_specs=pl.BlockSpec((BM, BN), lambda i, j: (i, j)),
        out_shape=jax.ShapeDtypeStruct((TB, FH), jnp.bfloat16),
        compiler_params=pltpu.CompilerParams(
            dimension_semantics=("parallel", "arbitrary")),
    )(x_tm, wxt, b2)

    wat = Wa.T.astype(jnp.bfloat16)

    out_bf = jax.ShapeDtypeStruct((B, T * H), jnp.bfloat16)
    out_spec = pl.BlockSpec((B, U * H), lambda t: (0, t))

    outs = pl.pallas_call(
        _lstm_step_kernel,
        grid=(T // U,),
        in_specs=[
            pl.BlockSpec((U * B, FH), lambda t: (t, 0)),
            pl.BlockSpec((H, FH), lambda t: (0, 0)),
            pl.BlockSpec((B, H), lambda t: (0, 0)),
            pl.BlockSpec((B, H), lambda t: (0, 0)),
        ],
        out_specs=[out_spec] * 6,
        out_shape=[out_bf] * 6,
        scratch_shapes=[
            pltpu.VMEM((B, H), jnp.float32),
            pltpu.VMEM((B, H), jnp.float32),
        ],
        compiler_params=pltpu.CompilerParams(
            dimension_semantics=("arbitrary",)),
    )(yx, wat, a0, c0)

    a, c, yi, yf, yg, yo = _to_bth_tc(list(outs))
    return (a, c, yi, yf, yg, yo)
